# baseline (speedup 1.0000x reference)
import jax
import jax.numpy as jnp
from jax import lax
from jax.experimental import pallas as pl
from jax.experimental.pallas import tpu as pltpu

B = 16384
D = 128
H = 64
C = 64
CHUNK = 4096
NSTEPS = B // CHUNK
MOM = 0.5

_BF = jnp.bfloat16


def _dot16(a, b, dims):
    return lax.dot_general(a.astype(_BF), b.astype(_BF), dims,
                           preferred_element_type=jnp.float32)


def _tc_body(xs_ref, cs_ref, xt_ref, ct_ref, w1_ref, w2_ref, b1_ref, g_ref,
             be_ref, b2_ref, c3_ref, tc_ref, lbl_ref, out_ref,
             h_s, h_t, s_s, s_t, n_s, n_t, m1_s, m1_t, m2_s, m2_t,
             mu_s, mu_t, dn_s, dn_t, tab):
    p = pl.program_id(0)
    c = pl.program_id(1)

    @pl.when((p == 0) & (c == 0))
    def _init():
        s_s[...] = jnp.zeros_like(s_s)
        s_t[...] = jnp.zeros_like(s_t)
        n_s[...] = jnp.zeros_like(n_s)
        n_t[...] = jnp.zeros_like(n_t)
        m1_s[...] = jnp.zeros_like(m1_s)
        m1_t[...] = jnp.zeros_like(m1_t)
        m2_s[...] = jnp.zeros_like(m2_s)
        m2_t[...] = jnp.zeros_like(m2_t)

    @pl.when(p == 0)
    def _phase0():
        def layer1(x_ref, h_scr, m1_acc):
            x = x_ref[...]
            h = _dot16(x, w1_ref[...], (((1,), (1,)), ((), ())))
            h = jnp.maximum(h + b1_ref[...], 0.0)
            h_scr[pl.ds(c * CHUNK, CHUNK), :] = h
            m1_acc[...] += jnp.sum(h, axis=0, keepdims=True)
        layer1(xs_ref, h_s, m1_s)
        layer1(xt_ref, h_t, m1_t)

    @pl.when((p == 1) & (c == 0))
    def _mean():
        mu_s[...] = m1_s[...] * (1.0 / B)
        mu_t[...] = m1_t[...] * (1.0 / B)

    @pl.when(p == 1)
    def _phase1():
        def sqdev(h_scr, mu, m2_acc):
            d = h_scr[pl.ds(c * CHUNK, CHUNK), :] - mu[...]
            m2_acc[...] += jnp.sum(d * d, axis=0, keepdims=True)
        sqdev(h_s, mu_s, m2_s)
        sqdev(h_t, mu_t, m2_t)

    @pl.when((p == 2) & (c == 0))
    def _denom():
        dn_s[...] = jnp.sqrt(m2_s[...] * (1.0 / B) + 1e-5)
        dn_t[...] = jnp.sqrt(m2_t[...] * (1.0 / B) + 1e-5)

    @pl.when(p == 2)
    def _phase2():
        ones = jnp.ones((CHUNK, 1), dtype=_BF)

        def layer2(h_scr, cl_ref, mu, dn, s_acc, n_acc):
            h = h_scr[pl.ds(c * CHUNK, CHUNK), :]
            hn = (h - mu[...]) / dn[...] * g_ref[...] + be_ref[...]
            f = _dot16(hn, w2_ref[...], (((1,), (1,)), ((), ())))
            f = f + b2_ref[...]
            ids = cl_ref[0]
            iota = lax.broadcasted_iota(jnp.int32, (C, CHUNK), 0)
            onehot = (ids == iota).astype(_BF)
            s_acc[...] += lax.dot_general(
                onehot, f.astype(_BF), (((1,), (0,)), ((), ())),
                preferred_element_type=jnp.float32)
            n_acc[...] += lax.dot_general(
                onehot, ones, (((1,), (0,)), ((), ())),
                preferred_element_type=jnp.float32)
        layer2(h_s, cs_ref, mu_s, dn_s, s_s, n_s)
        layer2(h_t, ct_ref, mu_t, dn_t, s_t, n_t)

    @pl.when((p == 2) & (c == NSTEPS - 1))
    def _tail():
        def centers(s_acc, n_acc, old):
            cnt = n_acc[...] + 1e-6
            m = 1.0 / cnt + 1.0
            m16 = m.astype(_BF).astype(jnp.float32)
            s16 = s_acc[...].astype(_BF).astype(jnp.float32)
            new = m16 * s16
            upd = MOM * old[...] + (1.0 - MOM) * new
            nrm = jnp.sqrt(jnp.sum(upd * upd, axis=1, keepdims=True))
            return upd / jnp.maximum(nrm, 1e-12)

        src_cc = centers(s_s, n_s, c3_ref)
        tgt_cc = centers(s_t, n_t, tc_ref)
        sim = _dot16(tgt_cc, src_cc, (((1,), (1,)), ((), ())))
        top = jnp.argmax(sim, axis=1, keepdims=True)
        iota = lax.broadcasted_iota(jnp.int32, (C, C), 1)
        oh_top = (top == iota).astype(jnp.float32)
        lbl = lbl_ref[...].astype(jnp.float32)
        table = lax.dot_general(oh_top, lbl, (((1,), (1,)), ((), ())),
                                preferred_element_type=jnp.float32)
        tab[...] = table.astype(_BF).reshape(1, C)

    @pl.when(p == 3)
    def _phase3():
        ids = ct_ref[0]
        iota = lax.broadcasted_iota(jnp.int32, (C, CHUNK), 0)
        onehot = (ids == iota).astype(_BF)
        vals = lax.dot_general(tab[...], onehot, (((1,), (0,)), ((), ())),
                               preferred_element_type=jnp.float32)
        out_ref[...] = vals.astype(jnp.int32)


def _tc_run(src_feat, src_cl3, tgt_feat, tgt_cl3, W1, W2, b1r, gr, ber,
            b2r, center3, tgt_centers, lbl3):
    grid = (4, NSTEPS)
    fspec = pl.BlockSpec((CHUNK, D),
                         lambda p, c: (jnp.where(p == 0, c, NSTEPS - 1), 0))
    cspec = pl.BlockSpec((1, 1, CHUNK),
                         lambda p, c: (jnp.where(p >= 2, c, 0), 0, 0))
    full = lambda shape: pl.BlockSpec(shape, lambda p, c: tuple(0 for _ in shape))
    return pl.pallas_call(
        _tc_body,
        grid=grid,
        in_specs=[
            fspec, cspec, fspec, cspec,
            full((H, D)), full((D, H)), full((1, H)), full((1, H)),
            full((1, H)), full((1, D)), full((C, D)), full((C, D)),
            full((1, C)),
        ],
        out_specs=pl.BlockSpec((1, CHUNK),
                               lambda p, c: (0, jnp.where(p == 3, c, 0))),
        out_shape=jax.ShapeDtypeStruct((1, B), jnp.int32),
        scratch_shapes=[
            pltpu.VMEM((B, H), jnp.float32), pltpu.VMEM((B, H), jnp.float32),
            pltpu.VMEM((C, D), jnp.float32), pltpu.VMEM((C, D), jnp.float32),
            pltpu.VMEM((C, 1), jnp.float32), pltpu.VMEM((C, 1), jnp.float32),
            pltpu.VMEM((1, H), jnp.float32), pltpu.VMEM((1, H), jnp.float32),
            pltpu.VMEM((1, H), jnp.float32), pltpu.VMEM((1, H), jnp.float32),
            pltpu.VMEM((1, H), jnp.float32), pltpu.VMEM((1, H), jnp.float32),
            pltpu.VMEM((1, H), jnp.float32), pltpu.VMEM((1, H), jnp.float32),
            pltpu.VMEM((1, C), _BF),
        ],
    )(src_feat, src_cl3, tgt_feat, tgt_cl3, W1, W2, b1r, gr, ber, b2r,
      center3, tgt_centers, lbl3)


def kernel(src_feat, src_cluster, src_idx, tgt_feat, tgt_cluster,
           src_cluster_labels, src_cluster_centers, tgt_cluster_centers,
           W1, b1, gamma, beta, W2, b2):
    center3 = lax.dynamic_index_in_dim(src_cluster_centers, src_idx, 0,
                                       keepdims=False)
    lbl3 = lax.dynamic_index_in_dim(src_cluster_labels, src_idx, 0,
                                    keepdims=True)
    src_cl3 = src_cluster.reshape(NSTEPS, 1, CHUNK)
    tgt_cl3 = tgt_cluster.reshape(NSTEPS, 1, CHUNK)
    out = _tc_run(
        src_feat, src_cl3, tgt_feat, tgt_cl3, W1, W2,
        b1.reshape(1, H), gamma.reshape(1, H), beta.reshape(1, H),
        b2.reshape(1, D), center3, tgt_cluster_centers, lbl3)
    return out.reshape(B)


# speedup vs baseline: 1.0911x; 1.0911x over previous
import jax
import jax.numpy as jnp
from jax import lax
from jax.experimental import pallas as pl
from jax.experimental.pallas import tpu as pltpu

B = 16384
D = 128
H = 64
C = 64
CHUNK = 8192
NSTEPS = B // CHUNK
MOM = 0.5

_BF = jnp.bfloat16


def _dot16(a, b, dims):
    return lax.dot_general(a.astype(_BF), b.astype(_BF), dims,
                           preferred_element_type=jnp.float32)


def _tc_body(xs_ref, cs_ref, xt_ref, ct_ref, w1_ref, w2_ref, b1_ref, g_ref,
             be_ref, b2_ref, c3_ref, tc_ref, lbl_ref, out_ref,
             h_s, h_t, s_s, s_t, n_s, n_t, m1_s, m1_t, m2_s, m2_t,
             mu_s, mu_t, dn_s, dn_t, tab):
    p = pl.program_id(0)
    c = pl.program_id(1)

    @pl.when((p == 0) & (c == 0))
    def _init():
        s_s[...] = jnp.zeros_like(s_s)
        s_t[...] = jnp.zeros_like(s_t)
        n_s[...] = jnp.zeros_like(n_s)
        n_t[...] = jnp.zeros_like(n_t)
        m1_s[...] = jnp.zeros_like(m1_s)
        m1_t[...] = jnp.zeros_like(m1_t)
        m2_s[...] = jnp.zeros_like(m2_s)
        m2_t[...] = jnp.zeros_like(m2_t)

    @pl.when(p == 0)
    def _phase0():
        def layer1(x_ref, h_scr, m1_acc):
            x = x_ref[...]
            h = _dot16(x, w1_ref[...], (((1,), (1,)), ((), ())))
            h = jnp.maximum(h + b1_ref[...], 0.0)
            h_scr[pl.ds(c * CHUNK, CHUNK), :] = h
            m1_acc[...] += jnp.sum(h, axis=0, keepdims=True)
        layer1(xs_ref, h_s, m1_s)
        layer1(xt_ref, h_t, m1_t)

    @pl.when((p == 1) & (c == 0))
    def _mean():
        mu_s[...] = m1_s[...] * (1.0 / B)
        mu_t[...] = m1_t[...] * (1.0 / B)

    @pl.when(p == 1)
    def _phase1():
        def sqdev(h_scr, mu, m2_acc):
            d = h_scr[pl.ds(c * CHUNK, CHUNK), :] - mu[...]
            m2_acc[...] += jnp.sum(d * d, axis=0, keepdims=True)
        sqdev(h_s, mu_s, m2_s)
        sqdev(h_t, mu_t, m2_t)

    @pl.when((p == 2) & (c == 0))
    def _denom():
        dn_s[...] = jnp.sqrt(m2_s[...] * (1.0 / B) + 1e-5)
        dn_t[...] = jnp.sqrt(m2_t[...] * (1.0 / B) + 1e-5)

    @pl.when(p == 2)
    def _phase2():
        ones = jnp.ones((CHUNK, 1), dtype=_BF)

        def layer2(h_scr, cl_ref, mu, dn, s_acc, n_acc):
            h = h_scr[pl.ds(c * CHUNK, CHUNK), :]
            hn = (h - mu[...]) / dn[...] * g_ref[...] + be_ref[...]
            f = _dot16(hn, w2_ref[...], (((1,), (1,)), ((), ())))
            f = f + b2_ref[...]
            ids = cl_ref[0]
            iota = lax.broadcasted_iota(jnp.int32, (C, CHUNK), 0)
            onehot = (ids == iota).astype(_BF)
            s_acc[...] += lax.dot_general(
                onehot, f.astype(_BF), (((1,), (0,)), ((), ())),
                preferred_element_type=jnp.float32)
            n_acc[...] += lax.dot_general(
                onehot, ones, (((1,), (0,)), ((), ())),
                preferred_element_type=jnp.float32)
        layer2(h_s, cs_ref, mu_s, dn_s, s_s, n_s)
        layer2(h_t, ct_ref, mu_t, dn_t, s_t, n_t)

    @pl.when((p == 2) & (c == NSTEPS - 1))
    def _tail():
        def centers(s_acc, n_acc, old):
            cnt = n_acc[...] + 1e-6
            m = 1.0 / cnt + 1.0
            m16 = m.astype(_BF).astype(jnp.float32)
            s16 = s_acc[...].astype(_BF).astype(jnp.float32)
            new = m16 * s16
            upd = MOM * old[...] + (1.0 - MOM) * new
            nrm = jnp.sqrt(jnp.sum(upd * upd, axis=1, keepdims=True))
            return upd / jnp.maximum(nrm, 1e-12)

        src_cc = centers(s_s, n_s, c3_ref)
        tgt_cc = centers(s_t, n_t, tc_ref)
        sim = _dot16(tgt_cc, src_cc, (((1,), (1,)), ((), ())))
        top = jnp.argmax(sim, axis=1, keepdims=True)
        iota = lax.broadcasted_iota(jnp.int32, (C, C), 1)
        oh_top = (top == iota).astype(jnp.float32)
        lbl = lbl_ref[...].astype(jnp.float32)
        table = lax.dot_general(oh_top, lbl, (((1,), (1,)), ((), ())),
                                preferred_element_type=jnp.float32)
        tab[...] = table.astype(_BF).reshape(1, C)

    @pl.when(p == 3)
    def _phase3():
        ids = ct_ref[0]
        iota = lax.broadcasted_iota(jnp.int32, (C, CHUNK), 0)
        onehot = (ids == iota).astype(_BF)
        vals = lax.dot_general(tab[...], onehot, (((1,), (0,)), ((), ())),
                               preferred_element_type=jnp.float32)
        out_ref[...] = vals.astype(jnp.int32)


def _tc_run(src_feat, src_cl3, tgt_feat, tgt_cl3, W1, W2, b1r, gr, ber,
            b2r, center3, tgt_centers, lbl3):
    grid = (4, NSTEPS)
    fspec = pl.BlockSpec((CHUNK, D),
                         lambda p, c: (jnp.where(p == 0, c, NSTEPS - 1), 0))
    cspec = pl.BlockSpec((1, 1, CHUNK),
                         lambda p, c: (jnp.where(p >= 2, c, 0), 0, 0))
    full = lambda shape: pl.BlockSpec(shape, lambda p, c: tuple(0 for _ in shape))
    return pl.pallas_call(
        _tc_body,
        grid=grid,
        in_specs=[
            fspec, cspec, fspec, cspec,
            full((H, D)), full((D, H)), full((1, H)), full((1, H)),
            full((1, H)), full((1, D)), full((C, D)), full((C, D)),
            full((1, C)),
        ],
        out_specs=pl.BlockSpec((1, CHUNK),
                               lambda p, c: (0, jnp.where(p == 3, c, 0))),
        out_shape=jax.ShapeDtypeStruct((1, B), jnp.int32),
        scratch_shapes=[
            pltpu.VMEM((B, H), jnp.float32), pltpu.VMEM((B, H), jnp.float32),
            pltpu.VMEM((C, D), jnp.float32), pltpu.VMEM((C, D), jnp.float32),
            pltpu.VMEM((C, 1), jnp.float32), pltpu.VMEM((C, 1), jnp.float32),
            pltpu.VMEM((1, H), jnp.float32), pltpu.VMEM((1, H), jnp.float32),
            pltpu.VMEM((1, H), jnp.float32), pltpu.VMEM((1, H), jnp.float32),
            pltpu.VMEM((1, H), jnp.float32), pltpu.VMEM((1, H), jnp.float32),
            pltpu.VMEM((1, H), jnp.float32), pltpu.VMEM((1, H), jnp.float32),
            pltpu.VMEM((1, C), _BF),
        ],
    )(src_feat, src_cl3, tgt_feat, tgt_cl3, W1, W2, b1r, gr, ber, b2r,
      center3, tgt_centers, lbl3)


def kernel(src_feat, src_cluster, src_idx, tgt_feat, tgt_cluster,
           src_cluster_labels, src_cluster_centers, tgt_cluster_centers,
           W1, b1, gamma, beta, W2, b2):
    center3 = lax.dynamic_index_in_dim(src_cluster_centers, src_idx, 0,
                                       keepdims=False)
    lbl3 = lax.dynamic_index_in_dim(src_cluster_labels, src_idx, 0,
                                    keepdims=True)
    src_cl3 = src_cluster.reshape(NSTEPS, 1, CHUNK)
    tgt_cl3 = tgt_cluster.reshape(NSTEPS, 1, CHUNK)
    out = _tc_run(
        src_feat, src_cl3, tgt_feat, tgt_cl3, W1, W2,
        b1.reshape(1, H), gamma.reshape(1, H), beta.reshape(1, H),
        b2.reshape(1, D), center3, tgt_cluster_centers, lbl3)
    return out.reshape(B)
